# baseline (speedup 1.0000x reference)
import functools

import jax
import jax.numpy as jnp
from jax import lax
from jax.experimental import pallas as pl
from jax.experimental.pallas import tpu as pltpu
from jax.experimental.pallas import tpu_sc as plsc

_BATCH = 4
_SEQ = 8192
_D = 1024
_NC = 2
_NS = 16
_NW = _NC * _NS
_SEQ_PER_W = _SEQ // _NW
_C = 32
_NCHUNK = _SEQ_PER_W // _C
_NVEC = _D // 16


def _sc_add(x2d, pe):
    mesh = plsc.VectorSubcoreMesh(core_axis_name="c", subcore_axis_name="s")

    @functools.partial(
        pl.kernel,
        mesh=mesh,
        out_type=jax.ShapeDtypeStruct((_BATCH * _SEQ, _D), jnp.float32),
        scratch_types=[
            pltpu.VMEM((_C, _D), jnp.float32),
            pltpu.VMEM((_C, _D), jnp.float32),
        ],
    )
    def k(x_hbm, pe_hbm, out_hbm, x_v, pe_v):
        wid = lax.axis_index("s") * _NC + lax.axis_index("c")
        seq0 = wid * _SEQ_PER_W

        def chunk_body(c, carry):
            base = seq0 + c * _C
            pltpu.sync_copy(pe_hbm.at[pl.ds(base, _C)], pe_v)

            def batch_body(b, carry):
                row0 = b * _SEQ + base
                pltpu.sync_copy(x_hbm.at[pl.ds(row0, _C)], x_v)

                def row_body(r, carry):
                    def vec_body(j, carry):
                        sl = pl.ds(j * 16, 16)
                        plsc.addupdate(x_v.at[r, sl], pe_v[r, sl])
                        return carry

                    return lax.fori_loop(0, _NVEC, vec_body, carry)

                lax.fori_loop(0, _C, row_body, 0)
                pltpu.sync_copy(x_v, out_hbm.at[pl.ds(row0, _C)])
                return carry

            lax.fori_loop(0, _BATCH, batch_body, 0)
            return carry

        lax.fori_loop(0, _NCHUNK, chunk_body, 0)

    return k(x2d, pe)


def kernel(x, pos_emb):
    b, s, d = x.shape
    x2d = x.reshape(b * s, d)
    out = _sc_add(x2d, pos_emb[:s])
    return out.reshape(b, s, d)


# speedup vs baseline: 1.2205x; 1.2205x over previous
import functools

import jax
import jax.numpy as jnp
from jax import lax
from jax.experimental import pallas as pl
from jax.experimental.pallas import tpu as pltpu
from jax.experimental.pallas import tpu_sc as plsc

_BATCH = 4
_SEQ = 8192
_D = 1024
_NC = 2
_NS = 16
_NW = _NC * _NS
_SEQ_PER_W = _SEQ // _NW
_C = 32
_NCHUNK = _SEQ_PER_W // _C
_NVEC = _D // 16


def _sc_add(x2d, pe):
    mesh = plsc.VectorSubcoreMesh(core_axis_name="c", subcore_axis_name="s")

    @functools.partial(
        pl.kernel,
        mesh=mesh,
        out_type=jax.ShapeDtypeStruct((_BATCH * _SEQ, _D), jnp.float32),
        scratch_types=[
            pltpu.VMEM((_C, _D), jnp.float32),
            pltpu.VMEM((_C, _D), jnp.float32),
        ],
    )
    def k(x_hbm, pe_hbm, out_hbm, x_v, pe_v):
        wid = lax.axis_index("s") * _NC + lax.axis_index("c")
        seq0 = wid * _SEQ_PER_W

        def chunk_body(c, carry):
            base = seq0 + c * _C
            pltpu.sync_copy(pe_hbm.at[pl.ds(base, _C)], pe_v)

            def batch_body(b, carry):
                row0 = b * _SEQ + base
                pltpu.sync_copy(x_hbm.at[pl.ds(row0, _C)], x_v)

                def row_body(r, carry):
                    for j in range(_NVEC):
                        sl = pl.ds(j * 16, 16)
                        plsc.addupdate(x_v.at[r, sl], pe_v[r, sl])
                    return carry

                lax.fori_loop(0, _C, row_body, 0)
                pltpu.sync_copy(x_v, out_hbm.at[pl.ds(row0, _C)])
                return carry

            lax.fori_loop(0, _BATCH, batch_body, 0)
            return carry

        lax.fori_loop(0, _NCHUNK, chunk_body, 0)

    return k(x2d, pe)


def kernel(x, pos_emb):
    b, s, d = x.shape
    x2d = x.reshape(b * s, d)
    out = _sc_add(x2d, pos_emb[:s])
    return out.reshape(b, s, d)


# speedup vs baseline: 2.7485x; 2.2519x over previous
import functools

import jax
import jax.numpy as jnp
from jax import lax
from jax.experimental import pallas as pl
from jax.experimental.pallas import tpu as pltpu
from jax.experimental.pallas import tpu_sc as plsc

_BATCH = 4
_SEQ = 8192
_D = 1024
_NC = 2
_NS = 16
_NW = _NC * _NS
_SEQ_PER_W = _SEQ // _NW
_C = 8
_NCHUNK = _SEQ_PER_W // _C
_NITEM = _NCHUNK * _BATCH
_NBUF = 4
_NGRP = _NITEM // _NBUF
_NVEC = _D // 16
_LOOKAHEAD = 2


def _sc_add(x2d, pe):
    mesh = plsc.VectorSubcoreMesh(core_axis_name="c", subcore_axis_name="s")

    scratch = (
        [pltpu.VMEM((_C, _D), jnp.float32) for _ in range(_NBUF)]
        + [pltpu.VMEM((_C, _D), jnp.float32) for _ in range(_NBUF)]
        + [pltpu.SemaphoreType.DMA for _ in range(2 * _NBUF)]
    )

    @functools.partial(
        pl.kernel,
        mesh=mesh,
        out_type=jax.ShapeDtypeStruct((_BATCH * _SEQ, _D), jnp.float32),
        scratch_types=scratch,
    )
    def k(x_hbm, pe_hbm, out_hbm, *bufs):
        x_v = bufs[:_NBUF]
        pe_v = bufs[_NBUF:2 * _NBUF]
        in_sem = bufs[2 * _NBUF:3 * _NBUF]
        out_sem = bufs[3 * _NBUF:]

        wid = lax.axis_index("s") * _NC + lax.axis_index("c")
        seq0 = wid * _SEQ_PER_W

        def item_rows(i):
            c = i // _BATCH
            b = i - c * _BATCH
            base = seq0 + c * _C
            return b * _SEQ + base, base

        def start_in(j, slot):
            xrow, perow = item_rows(j)
            pltpu.async_copy(x_hbm.at[pl.ds(xrow, _C)], x_v[slot], in_sem[slot])
            pltpu.async_copy(pe_hbm.at[pl.ds(perow, _C)], pe_v[slot], in_sem[slot])

        def wait_in(j, slot):
            xrow, perow = item_rows(j)
            pltpu.make_async_copy(
                x_hbm.at[pl.ds(xrow, _C)], x_v[slot], in_sem[slot]).wait()
            pltpu.make_async_copy(
                pe_hbm.at[pl.ds(perow, _C)], pe_v[slot], in_sem[slot]).wait()

        def start_out(j, slot):
            xrow, _ = item_rows(j)
            pltpu.async_copy(x_v[slot], out_hbm.at[pl.ds(xrow, _C)], out_sem[slot])

        def wait_out(j, slot):
            xrow, _ = item_rows(j)
            pltpu.make_async_copy(
                x_v[slot], out_hbm.at[pl.ds(xrow, _C)], out_sem[slot]).wait()

        for j in range(_LOOKAHEAD):
            start_in(j, j % _NBUF)

        def group(g, carry):
            for slot in range(_NBUF):
                i = g * _NBUF + slot
                j = i + _LOOKAHEAD
                jslot = (slot + _LOOKAHEAD) % _NBUF

                @pl.when(j < _NITEM)
                def _():
                    @pl.when(j >= _NBUF)
                    def _():
                        wait_out(j - _NBUF, jslot)

                    start_in(j, jslot)

                wait_in(i, slot)

                def row_body(r, carry):
                    for v in range(_NVEC):
                        sl = pl.ds(v * 16, 16)
                        plsc.addupdate(x_v[slot].at[r, sl], pe_v[slot][r, sl])
                    return carry

                lax.fori_loop(0, _C, row_body, 0)
                start_out(i, slot)
            return carry

        lax.fori_loop(0, _NGRP, group, 0)

        for slot in range(_NBUF):
            wait_out(_NITEM - _NBUF + slot, slot)

    return k(x2d, pe)


def kernel(x, pos_emb):
    b, s, d = x.shape
    x2d = x.reshape(b * s, d)
    out = _sc_add(x2d, pos_emb[:s])
    return out.reshape(b, s, d)


# speedup vs baseline: 3.4697x; 1.2624x over previous
import functools

import jax
import jax.numpy as jnp
from jax import lax
from jax.experimental import pallas as pl
from jax.experimental.pallas import tpu as pltpu
from jax.experimental.pallas import tpu_sc as plsc

_BATCH = 4
_SEQ = 8192
_D = 1024
_NC = 2
_NS = 16
_NW = _NC * _NS
_SEQ_PER_W = _SEQ // _NW
_C = 8
_NCHUNK = _SEQ_PER_W // _C
_NXB = 2 * _BATCH
_NPE = 2
_NVEC = _D // 16


def _sc_add(x2d, pe):
    mesh = plsc.VectorSubcoreMesh(core_axis_name="c", subcore_axis_name="s")

    scratch = (
        [pltpu.VMEM((_C, _D), jnp.float32) for _ in range(_NXB)]
        + [pltpu.VMEM((_C, _D), jnp.float32) for _ in range(_NPE)]
        + [pltpu.SemaphoreType.DMA for _ in range(_NXB + _NPE + _NXB)]
    )

    @functools.partial(
        pl.kernel,
        mesh=mesh,
        out_type=jax.ShapeDtypeStruct((_BATCH * _SEQ, _D), jnp.float32),
        scratch_types=scratch,
    )
    def k(x_hbm, pe_hbm, out_hbm, *bufs):
        x_v = bufs[:_NXB]
        pe_v = bufs[_NXB:_NXB + _NPE]
        sems = bufs[_NXB + _NPE:]
        in_sem = sems[:_NXB]
        pe_sem = sems[_NXB:_NXB + _NPE]
        out_sem = sems[_NXB + _NPE:]

        wid = lax.axis_index("s") * _NC + lax.axis_index("c")
        seq0 = wid * _SEQ_PER_W

        def xrow(c, b):
            return b * _SEQ + seq0 + c * _C

        def perow(c):
            return seq0 + c * _C

        def start_x(c, b, slot):
            pltpu.async_copy(
                x_hbm.at[pl.ds(xrow(c, b), _C)], x_v[slot], in_sem[slot])

        def wait_x(c, b, slot):
            pltpu.make_async_copy(
                x_hbm.at[pl.ds(xrow(c, b), _C)], x_v[slot], in_sem[slot]).wait()

        def start_pe(c, s):
            pltpu.async_copy(
                pe_hbm.at[pl.ds(perow(c), _C)], pe_v[s], pe_sem[s])

        def wait_pe(c, s):
            pltpu.make_async_copy(
                pe_hbm.at[pl.ds(perow(c), _C)], pe_v[s], pe_sem[s]).wait()

        def start_out(c, b, slot):
            pltpu.async_copy(
                x_v[slot], out_hbm.at[pl.ds(xrow(c, b), _C)], out_sem[slot])

        def wait_out(c, b, slot):
            pltpu.make_async_copy(
                x_v[slot], out_hbm.at[pl.ds(xrow(c, b), _C)],
                out_sem[slot]).wait()

        def add_item(slot, ps):
            def row_body(r, carry):
                for v in range(_NVEC):
                    sl = pl.ds(v * 16, 16)
                    plsc.addupdate(x_v[slot].at[r, sl], pe_v[ps][r, sl])
                return carry
            lax.fori_loop(0, _C, row_body, 0)

        start_pe(0, 0)
        start_pe(1, 1)
        for b in range(_BATCH):
            start_x(0, b, b)
        for b in range(_BATCH):
            start_x(1, b, _BATCH + b)

        def iter_body(t, carry):
            c0 = 2 * t
            c1 = c0 + 1

            wait_pe(c0, 0)
            for b in range(_BATCH):
                @pl.when(t > 0)
                def _():
                    wait_out(c0 - 1, b, _BATCH + b)
                    start_x(c1, b, _BATCH + b)

                wait_x(c0, b, b)
                add_item(b, 0)
                start_out(c0, b, b)

            @pl.when(c0 + 2 < _NCHUNK)
            def _():
                start_pe(c0 + 2, 0)

            wait_pe(c1, 1)
            for b in range(_BATCH):
                wait_out(c0, b, b)

                @pl.when(c1 + 1 < _NCHUNK)
                def _():
                    start_x(c1 + 1, b, b)

                wait_x(c1, b, _BATCH + b)
                add_item(_BATCH + b, 1)
                start_out(c1, b, _BATCH + b)

            @pl.when(c1 + 2 < _NCHUNK)
            def _():
                start_pe(c1 + 2, 1)

            return carry

        lax.fori_loop(0, _NCHUNK // 2, iter_body, 0)

        for b in range(_BATCH):
            wait_out(_NCHUNK - 1, b, _BATCH + b)

    return k(x2d, pe)


def kernel(x, pos_emb):
    b, s, d = x.shape
    x2d = x.reshape(b * s, d)
    out = _sc_add(x2d, pos_emb[:s])
    return out.reshape(b, s, d)
